# baseline (speedup 1.0000x reference)
import jax
import jax.numpy as jnp
from jax import lax
from jax.experimental import pallas as pl
from jax.experimental.pallas import tpu as pltpu
from jax.experimental.pallas import tpu_sc as plsc

N_NODES = 10000
N_EDGES = 320000
D = 128
H = 2
DH = 64
E_TOT = N_EDGES + N_NODES

NC = 2
NS = 16
NW = NC * NS
CHUNK = 128
ROWS_PER_TILE = 82
EP_ROWS = NW * ROWS_PER_TILE
EP = EP_ROWS * CHUNK
ACC_W = 64
WB = 624
WB_TAIL = N_NODES - NS * WB
DEN_ROWS = 640


def _prep_body(f_ref, wl_ref, wa_ref, a_ref, val0_ref, val1_ref,
               alpha_ref, k_ref):
    f = f_ref[...]
    val = jnp.dot(f, wl_ref[...].T, preferred_element_type=jnp.float32)
    val0_ref[...] = val[:, 0:DH]
    val1_ref[...] = val[:, DH:2 * DH]
    wa = wa_ref[...]
    av = a_ref[...]
    c0 = jnp.dot(av[0:1, 0:DH], wa[0:DH, :], preferred_element_type=jnp.float32)
    c1 = jnp.dot(av[1:2, 0:DH], wa[DH:2 * DH, :], preferred_element_type=jnp.float32)
    d0 = jnp.dot(av[0:1, DH:2 * DH], wa[0:DH, :], preferred_element_type=jnp.float32)
    d1 = jnp.dot(av[1:2, DH:2 * DH], wa[DH:2 * DH, :], preferred_element_type=jnp.float32)
    cm = jnp.concatenate([c0, c1, d0, d1], axis=0)
    alpha = jnp.dot(f, cm.T, preferred_element_type=jnp.float32)
    alpha_ref[0] = jnp.concatenate([alpha[:, 0:1], alpha[:, 2:3]], axis=1)
    alpha_ref[1] = jnp.concatenate([alpha[:, 1:2], alpha[:, 3:4]], axis=1)
    amax = jnp.max(alpha, axis=0, keepdims=True)
    k0 = jnp.maximum(amax[0, 0] + amax[0, 2], 0.0)
    k1 = jnp.maximum(amax[0, 1] + amax[0, 3], 0.0)
    col = lax.broadcasted_iota(jnp.int32, (8, 128), 1)
    row = lax.broadcasted_iota(jnp.int32, (8, 128), 0)
    k_ref[...] = jnp.where((row == 0) & (col == 0), k0,
                           jnp.where((row == 0) & (col == 1), k1, 0.0))


def _prep(feature, w_lin, w_attn, a):
    return pl.pallas_call(
        _prep_body,
        out_shape=[
            jax.ShapeDtypeStruct((N_NODES, DH), jnp.float32),
            jax.ShapeDtypeStruct((N_NODES, DH), jnp.float32),
            jax.ShapeDtypeStruct((H, N_NODES, 2), jnp.float32),
            jax.ShapeDtypeStruct((8, 128), jnp.float32),
        ],
    )(feature, w_lin, w_attn, a)


def _edge_body(val0_hbm, val1_hbm, alpha_hbm, src_hbm, dst_hbm, k_hbm,
               out_hbm, outd_hbm, alpha_v, kv, srcall, dstall, rows_a, rows_b,
               scat_a, scat_b, pb_a, pb_b, den_v, iox,
               acc, acc_den, gsem_a, gsem_b, ssem_a, ssem_b):
    cid = lax.axis_index("c")
    sid = lax.axis_index("s")
    wid = cid * NS + sid

    pltpu.sync_copy(k_hbm, kv)
    pltpu.sync_copy(src_hbm.at[pl.ds(wid * ROWS_PER_TILE, ROWS_PER_TILE)],
                    srcall)
    pltpu.sync_copy(dst_hbm.at[pl.ds(wid * ROWS_PER_TILE, ROWS_PER_TILE)],
                    dstall)
    kvec = kv[pl.ds(0, 16)]
    lane = lax.iota(jnp.int32, 16)

    for j in range(DEN_ROWS // CHUNK):
        for g in range(CHUNK // 16):
            iox[j, pl.ds(g * 16, 16)] = j * CHUNK + g * 16 + lane

    for h, val_hbm in ((0, val0_hbm), (1, val1_hbm)):
        kh = kvec[h]
        pltpu.sync_copy(alpha_hbm.at[h], alpha_v)

        def _zrow(r, _):
            for c in range(ACC_W // 16):
                scat_a[r, pl.ds(c * 16, 16)] = jnp.zeros((16,), jnp.float32)
            return 0
        lax.fori_loop(0, CHUNK, _zrow, 0)

        def _zden(r, _):
            den_v[r, pl.ds(0, 16)] = jnp.zeros((16,), jnp.float32)
            return 0
        lax.fori_loop(0, DEN_ROWS, _zden, 0)

        for j in range(4):
            pltpu.sync_copy(scat_a, acc.at[pl.ds(sid * WB + j * CHUNK, CHUNK)])
        pltpu.sync_copy(scat_a.at[pl.ds(0, WB - 4 * CHUNK)],
                        acc.at[pl.ds(sid * WB + 4 * CHUNK, WB - 4 * CHUNK)])

        @pl.when(sid == 0)
        def _zero_tail():
            pltpu.sync_copy(scat_a.at[pl.ds(0, WB_TAIL)],
                            acc.at[pl.ds(NS * WB, WB_TAIL)])

        pltpu.sync_copy(den_v.at[pl.ds(0, DEN_ROWS // NS)],
                        acc_den.at[pl.ds(sid * (DEN_ROWS // NS),
                                         DEN_ROWS // NS)])

        plsc.subcore_barrier()

        def _compute_p(r, pb):
            ebase = (wid * ROWS_PER_TILE + r) * CHUNK

            @plsc.parallel_loop(0, CHUNK // 16, unroll=2)
            def _pgrp(g):
                sv = srcall[r, pl.ds(g * 16, 16)]
                dv = dstall[r, pl.ds(g * 16, 16)]
                a_s = plsc.load_gather(alpha_v, [sv * 2])
                a_d = plsc.load_gather(alpha_v, [dv * 2 + 1])
                s = a_s + a_d
                s = jnp.where(s >= 0, s, 0.2 * s) - kh
                eid = ebase + g * 16 + lane
                valid = (sv != dv) | ((eid >= N_EDGES) & (eid < E_TOT))
                pb[pl.ds(g * 16, 16)] = jnp.where(valid, jnp.exp(s), 0.0)

        def _scale(rows_v, scat_v, pb):
            @plsc.parallel_loop(0, CHUNK // 16, unroll=2)
            def _grp(g2):
                pv = pb[pl.ds(g2 * 16, 16)]
                for j in range(16):
                    e = g2 * 16 + j
                    p = pv[j]
                    ab = rows_v[e, pl.ds(0, 32)]
                    cd = rows_v[e, pl.ds(32, 32)]
                    va, vb = plsc.unpack(
                        ab, format=plsc.PackFormat.INTERLEAVED,
                        preferred_element_type=jnp.float32)
                    vc, vd = plsc.unpack(
                        cd, format=plsc.PackFormat.INTERLEAVED,
                        preferred_element_type=jnp.float32)
                    scat_v[e, pl.ds(0, 16)] = va * p
                    scat_v[e, pl.ds(16, 16)] = vb * p
                    scat_v[e, pl.ds(32, 16)] = vc * p
                    scat_v[e, pl.ds(48, 16)] = vd * p

        def _den_add(r, pb):
            for g in range(CHUNK // 16):
                dv = dstall[r, pl.ds(g * 16, 16)]
                pvec = pb[pl.ds(g * 16, 16)]
                plsc.addupdate_scatter(
                    den_v, [dv >> 4, dv & 15], pvec)

        NB = ROWS_PER_TILE // 2

        pltpu.async_copy(val_hbm.at[srcall.at[0]], rows_a, gsem_a)
        pltpu.async_copy(val_hbm.at[srcall.at[1]], rows_b, gsem_b)

        def _iter(i, _):
            r0 = 2 * i
            r1 = 2 * i + 1
            _compute_p(r0, pb_a)
            _compute_p(r1, pb_b)
            _den_add(r0, pb_a)
            _den_add(r1, pb_b)

            pltpu.make_async_copy(
                val_hbm.at[srcall.at[r0]], rows_a, gsem_a).wait()

            @pl.when(i > 0)
            def _drain_a():
                pltpu.make_async_copy(
                    scat_a, acc.at[dstall.at[r0]], ssem_a).wait()

            _scale(rows_a, scat_a, pb_a)
            pltpu.async_copy(scat_a, acc.at[dstall.at[r0]], ssem_a, add=True)

            @pl.when(i < NB - 1)
            def _next_a():
                pltpu.async_copy(
                    val_hbm.at[srcall.at[r0 + 2]], rows_a, gsem_a)

            pltpu.make_async_copy(
                val_hbm.at[srcall.at[r1]], rows_b, gsem_b).wait()

            @pl.when(i > 0)
            def _drain_b():
                pltpu.make_async_copy(
                    scat_b, acc.at[dstall.at[r1]], ssem_b).wait()

            _scale(rows_b, scat_b, pb_b)
            pltpu.async_copy(scat_b, acc.at[dstall.at[r1]], ssem_b, add=True)

            @pl.when(i < NB - 1)
            def _next_b():
                pltpu.async_copy(
                    val_hbm.at[srcall.at[r1 + 2]], rows_b, gsem_b)

            return 0

        lax.fori_loop(0, NB, _iter, 0)
        pltpu.make_async_copy(scat_a, acc.at[dstall.at[0]], ssem_a).wait()
        pltpu.make_async_copy(scat_b, acc.at[dstall.at[1]], ssem_b).wait()

        for j in range(DEN_ROWS // CHUNK):
            pltpu.sync_copy(den_v.at[pl.ds(j * CHUNK, CHUNK)],
                            acc_den.at[iox.at[j]], add=True)

        plsc.subcore_barrier()

        pltpu.sync_copy(acc.at[pl.ds(sid * WB, WB)],
                        out_hbm.at[cid].at[h].at[pl.ds(sid * WB, WB)])
        pltpu.sync_copy(
            acc_den.at[pl.ds(sid * (DEN_ROWS // NS), DEN_ROWS // NS)],
            outd_hbm.at[cid].at[h].at[pl.ds(sid * (DEN_ROWS // NS),
                                            DEN_ROWS // NS)])

        @pl.when(sid == 0)
        def _tail():
            pltpu.sync_copy(acc.at[pl.ds(NS * WB, WB_TAIL)],
                            out_hbm.at[cid].at[h].at[pl.ds(NS * WB, WB_TAIL)])

        plsc.subcore_barrier()


def _edge_pass(val0, val1, alpha2, src_rows, dst_rows, k16):
    mesh = plsc.VectorSubcoreMesh(core_axis_name="c", subcore_axis_name="s")
    fn = pl.kernel(
        _edge_body,
        out_type=[
            jax.ShapeDtypeStruct((NC, H, N_NODES, ACC_W), jnp.float32),
            jax.ShapeDtypeStruct((NC, H, DEN_ROWS, 16), jnp.float32),
        ],
        mesh=mesh,
        compiler_params=pltpu.CompilerParams(
            use_tc_tiling_on_sc=False, needs_layout_passes=False),
        scratch_types=[
            pltpu.VMEM((N_NODES * 2,), jnp.float32),
            pltpu.VMEM((16,), jnp.float32),
            pltpu.VMEM((ROWS_PER_TILE, CHUNK), jnp.int32),
            pltpu.VMEM((ROWS_PER_TILE, CHUNK), jnp.int32),
            pltpu.VMEM((CHUNK, DH), jnp.bfloat16),
            pltpu.VMEM((CHUNK, DH), jnp.bfloat16),
            pltpu.VMEM((CHUNK, ACC_W), jnp.float32),
            pltpu.VMEM((CHUNK, ACC_W), jnp.float32),
            pltpu.VMEM((CHUNK,), jnp.float32),
            pltpu.VMEM((CHUNK,), jnp.float32),
            pltpu.VMEM((DEN_ROWS, 16), jnp.float32),
            pltpu.VMEM((DEN_ROWS // CHUNK, CHUNK), jnp.int32),
            pltpu.VMEM_SHARED((N_NODES, ACC_W), jnp.float32),
            pltpu.VMEM_SHARED((DEN_ROWS, 16), jnp.float32),
            pltpu.SemaphoreType.DMA,
            pltpu.SemaphoreType.DMA,
            pltpu.SemaphoreType.DMA,
            pltpu.SemaphoreType.DMA,
        ],
    )
    return fn(val0, val1, alpha2, src_rows, dst_rows, k16)


def _norm_body(acc_ref, den_ref, out_ref):
    s0 = acc_ref[0, 0] + acc_ref[1, 0]
    s1 = acc_ref[0, 1] + acc_ref[1, 1]
    d0 = den_ref[0, 0] + den_ref[1, 0]
    d1 = den_ref[0, 1] + den_ref[1, 1]
    out_ref[...] = jnp.concatenate(
        [s0 / d0[0:N_NODES], s1 / d1[0:N_NODES]], axis=1)


def _norm(acc, den):
    return pl.pallas_call(
        _norm_body,
        out_shape=jax.ShapeDtypeStruct((N_NODES, D), jnp.float32),
    )(acc, den)


def _perm_bf16(v):
    n = v.shape[0]
    return (v.reshape(n, 2, 2, 16).transpose(0, 1, 3, 2)
            .reshape(n, 2 * DH // 2).astype(jnp.bfloat16))


@jax.jit
def kernel(feature, edge_index, W_lin, W_attn, a):
    val0, val1, alpha, kmat = _prep(feature, W_lin, W_attn, a)
    alpha2 = alpha.reshape(H, 2 * N_NODES)
    k16 = kmat[0, :16]
    src0 = edge_index[0].astype(jnp.int32)
    dst0 = edge_index[1].astype(jnp.int32)
    loop = jnp.arange(N_NODES, dtype=jnp.int32)
    pad = jnp.zeros((EP - E_TOT,), jnp.int32)
    src_rows = jnp.concatenate([src0, loop, pad]).reshape(EP_ROWS, CHUNK)
    dst_rows = jnp.concatenate([dst0, loop, pad]).reshape(EP_ROWS, CHUNK)
    acc, den = _edge_pass(_perm_bf16(val0), _perm_bf16(val1), alpha2,
                          src_rows, dst_rows, k16)
    return _norm(acc, den.reshape(NC, H, DEN_ROWS * 16, 1))


# speedup vs baseline: 1.0185x; 1.0185x over previous
import jax
import jax.numpy as jnp
from jax import lax
from jax.experimental import pallas as pl
from jax.experimental.pallas import tpu as pltpu
from jax.experimental.pallas import tpu_sc as plsc

N_NODES = 10000
N_EDGES = 320000
D = 128
H = 2
DH = 64
E_TOT = N_EDGES + N_NODES

NC = 2
NS = 16
NW = NC * NS
CHUNK = 128
ROWS_PER_TILE = 82
EP_ROWS = NW * ROWS_PER_TILE
EP = EP_ROWS * CHUNK
ACC_W = 80
WB = 624
WB_TAIL = N_NODES - NS * WB


def _prep_body(f_ref, wl_ref, wa_ref, a_ref, val0_ref, val1_ref,
               alpha_ref, k_ref):
    f = f_ref[...]
    val = jnp.dot(f, wl_ref[...].T, preferred_element_type=jnp.float32)
    val0_ref[...] = val[:, 0:DH]
    val1_ref[...] = val[:, DH:2 * DH]
    wa = wa_ref[...]
    av = a_ref[...]
    c0 = jnp.dot(av[0:1, 0:DH], wa[0:DH, :], preferred_element_type=jnp.float32)
    c1 = jnp.dot(av[1:2, 0:DH], wa[DH:2 * DH, :], preferred_element_type=jnp.float32)
    d0 = jnp.dot(av[0:1, DH:2 * DH], wa[0:DH, :], preferred_element_type=jnp.float32)
    d1 = jnp.dot(av[1:2, DH:2 * DH], wa[DH:2 * DH, :], preferred_element_type=jnp.float32)
    cm = jnp.concatenate([c0, c1, d0, d1], axis=0)
    alpha = jnp.dot(f, cm.T, preferred_element_type=jnp.float32)
    alpha_ref[0] = jnp.concatenate([alpha[:, 0:1], alpha[:, 2:3]], axis=1)
    alpha_ref[1] = jnp.concatenate([alpha[:, 1:2], alpha[:, 3:4]], axis=1)
    amax = jnp.max(alpha, axis=0, keepdims=True)
    k0 = jnp.maximum(amax[0, 0] + amax[0, 2], 0.0)
    k1 = jnp.maximum(amax[0, 1] + amax[0, 3], 0.0)
    col = lax.broadcasted_iota(jnp.int32, (8, 128), 1)
    row = lax.broadcasted_iota(jnp.int32, (8, 128), 0)
    k_ref[...] = jnp.where((row == 0) & (col == 0), k0,
                           jnp.where((row == 0) & (col == 1), k1, 0.0))


def _prep(feature, w_lin, w_attn, a):
    return pl.pallas_call(
        _prep_body,
        out_shape=[
            jax.ShapeDtypeStruct((N_NODES, DH), jnp.float32),
            jax.ShapeDtypeStruct((N_NODES, DH), jnp.float32),
            jax.ShapeDtypeStruct((H, N_NODES, 2), jnp.float32),
            jax.ShapeDtypeStruct((8, 128), jnp.float32),
        ],
    )(feature, w_lin, w_attn, a)


def _edge_body(val0_hbm, val1_hbm, alpha_hbm, src_hbm, dst_hbm, k_hbm,
               out_hbm, alpha_v, kv, srcall, dstall, rows_a, rows_b,
               scat_a, scat_b, pb_a, pb_b,
               acc, gsem_a, gsem_b, ssem_a, ssem_b):
    cid = lax.axis_index("c")
    sid = lax.axis_index("s")
    wid = cid * NS + sid

    pltpu.sync_copy(k_hbm, kv)
    pltpu.sync_copy(src_hbm.at[pl.ds(wid * ROWS_PER_TILE, ROWS_PER_TILE)],
                    srcall)
    pltpu.sync_copy(dst_hbm.at[pl.ds(wid * ROWS_PER_TILE, ROWS_PER_TILE)],
                    dstall)
    kvec = kv[pl.ds(0, 16)]
    lane = lax.iota(jnp.int32, 16)

    for h, val_hbm in ((0, val0_hbm), (1, val1_hbm)):
        kh = kvec[h]
        pltpu.sync_copy(alpha_hbm.at[h], alpha_v)

        def _zrow(r, _):
            for c in range(ACC_W // 16):
                scat_a[r, pl.ds(c * 16, 16)] = jnp.zeros((16,), jnp.float32)
            return 0
        lax.fori_loop(0, CHUNK, _zrow, 0)

        for j in range(4):
            pltpu.sync_copy(scat_a, acc.at[pl.ds(sid * WB + j * CHUNK, CHUNK)])
        pltpu.sync_copy(scat_a.at[pl.ds(0, WB - 4 * CHUNK)],
                        acc.at[pl.ds(sid * WB + 4 * CHUNK, WB - 4 * CHUNK)])

        @pl.when(sid == 0)
        def _zero_tail():
            pltpu.sync_copy(scat_a.at[pl.ds(0, WB_TAIL)],
                            acc.at[pl.ds(NS * WB, WB_TAIL)])

        plsc.subcore_barrier()

        def _compute_p(r, pb):
            ebase = (wid * ROWS_PER_TILE + r) * CHUNK

            @plsc.parallel_loop(0, CHUNK // 16, unroll=2)
            def _pgrp(g):
                sv = srcall[r, pl.ds(g * 16, 16)]
                dv = dstall[r, pl.ds(g * 16, 16)]
                a_s = plsc.load_gather(alpha_v, [sv * 2])
                a_d = plsc.load_gather(alpha_v, [dv * 2 + 1])
                s = a_s + a_d
                s = jnp.where(s >= 0, s, 0.2 * s) - kh
                eid = ebase + g * 16 + lane
                valid = (sv != dv) | ((eid >= N_EDGES) & (eid < E_TOT))
                pb[pl.ds(g * 16, 16)] = jnp.where(valid, jnp.exp(s), 0.0)

        def _scale(rows_v, scat_v, pb):
            @plsc.parallel_loop(0, CHUNK // 16, unroll=2)
            def _grp(g2):
                pv = pb[pl.ds(g2 * 16, 16)]
                for j in range(16):
                    e = g2 * 16 + j
                    p = pv[j]
                    ab = rows_v[e, pl.ds(0, 32)]
                    cd = rows_v[e, pl.ds(32, 32)]
                    va, vb = plsc.unpack(
                        ab, format=plsc.PackFormat.INTERLEAVED,
                        preferred_element_type=jnp.float32)
                    vc, vd = plsc.unpack(
                        cd, format=plsc.PackFormat.INTERLEAVED,
                        preferred_element_type=jnp.float32)
                    scat_v[e, pl.ds(0, 16)] = va * p
                    scat_v[e, pl.ds(16, 16)] = vb * p
                    scat_v[e, pl.ds(32, 16)] = vc * p
                    scat_v[e, pl.ds(48, 16)] = vd * p
                    scat_v[e, pl.ds(DH, 16)] = jnp.where(lane == 0, p, 0.0)

        NB = ROWS_PER_TILE // 2

        pltpu.async_copy(val_hbm.at[srcall.at[0]], rows_a, gsem_a)
        pltpu.async_copy(val_hbm.at[srcall.at[1]], rows_b, gsem_b)

        def _iter(i, _):
            r0 = 2 * i
            r1 = 2 * i + 1
            _compute_p(r0, pb_a)
            _compute_p(r1, pb_b)

            pltpu.make_async_copy(
                val_hbm.at[srcall.at[r0]], rows_a, gsem_a).wait()

            @pl.when(i > 0)
            def _drain_a():
                pltpu.make_async_copy(
                    scat_a, acc.at[dstall.at[r0]], ssem_a).wait()

            _scale(rows_a, scat_a, pb_a)
            pltpu.async_copy(scat_a, acc.at[dstall.at[r0]], ssem_a, add=True)

            @pl.when(i < NB - 1)
            def _next_a():
                pltpu.async_copy(
                    val_hbm.at[srcall.at[r0 + 2]], rows_a, gsem_a)

            pltpu.make_async_copy(
                val_hbm.at[srcall.at[r1]], rows_b, gsem_b).wait()

            @pl.when(i > 0)
            def _drain_b():
                pltpu.make_async_copy(
                    scat_b, acc.at[dstall.at[r1]], ssem_b).wait()

            _scale(rows_b, scat_b, pb_b)
            pltpu.async_copy(scat_b, acc.at[dstall.at[r1]], ssem_b, add=True)

            @pl.when(i < NB - 1)
            def _next_b():
                pltpu.async_copy(
                    val_hbm.at[srcall.at[r1 + 2]], rows_b, gsem_b)

            return 0

        lax.fori_loop(0, NB, _iter, 0)
        pltpu.make_async_copy(scat_a, acc.at[dstall.at[0]], ssem_a).wait()
        pltpu.make_async_copy(scat_b, acc.at[dstall.at[1]], ssem_b).wait()
        plsc.subcore_barrier()

        pltpu.sync_copy(acc.at[pl.ds(sid * WB, WB)],
                        out_hbm.at[cid].at[h].at[pl.ds(sid * WB, WB)])

        @pl.when(sid == 0)
        def _tail():
            pltpu.sync_copy(acc.at[pl.ds(NS * WB, WB_TAIL)],
                            out_hbm.at[cid].at[h].at[pl.ds(NS * WB, WB_TAIL)])

        plsc.subcore_barrier()


def _edge_pass(val0, val1, alpha2, src_rows, dst_rows, k16):
    mesh = plsc.VectorSubcoreMesh(core_axis_name="c", subcore_axis_name="s")
    fn = pl.kernel(
        _edge_body,
        out_type=jax.ShapeDtypeStruct((NC, H, N_NODES, ACC_W), jnp.float32),
        mesh=mesh,
        compiler_params=pltpu.CompilerParams(
            use_tc_tiling_on_sc=False, needs_layout_passes=False),
        scratch_types=[
            pltpu.VMEM((N_NODES * 2,), jnp.float32),
            pltpu.VMEM((16,), jnp.float32),
            pltpu.VMEM((ROWS_PER_TILE, CHUNK), jnp.int32),
            pltpu.VMEM((ROWS_PER_TILE, CHUNK), jnp.int32),
            pltpu.VMEM((CHUNK, DH), jnp.bfloat16),
            pltpu.VMEM((CHUNK, DH), jnp.bfloat16),
            pltpu.VMEM((CHUNK, ACC_W), jnp.float32),
            pltpu.VMEM((CHUNK, ACC_W), jnp.float32),
            pltpu.VMEM((CHUNK,), jnp.float32),
            pltpu.VMEM((CHUNK,), jnp.float32),
            pltpu.VMEM_SHARED((N_NODES, ACC_W), jnp.float32),
            pltpu.SemaphoreType.DMA,
            pltpu.SemaphoreType.DMA,
            pltpu.SemaphoreType.DMA,
            pltpu.SemaphoreType.DMA,
        ],
    )
    return fn(val0, val1, alpha2, src_rows, dst_rows, k16)


def _norm_body(acc_ref, out_ref):
    s0 = acc_ref[0, 0] + acc_ref[1, 0]
    s1 = acc_ref[0, 1] + acc_ref[1, 1]
    out_ref[...] = jnp.concatenate(
        [s0[:, 0:DH] / s0[:, DH:DH + 1], s1[:, 0:DH] / s1[:, DH:DH + 1]],
        axis=1)


def _norm(acc):
    return pl.pallas_call(
        _norm_body,
        out_shape=jax.ShapeDtypeStruct((N_NODES, D), jnp.float32),
    )(acc)


def _perm_bf16(v):
    n = v.shape[0]
    return (v.reshape(n, 2, 2, 16).transpose(0, 1, 3, 2)
            .reshape(n, 2 * DH // 2).astype(jnp.bfloat16))


@jax.jit
def kernel(feature, edge_index, W_lin, W_attn, a):
    val0, val1, alpha, kmat = _prep(feature, W_lin, W_attn, a)
    alpha2 = alpha.reshape(H, 2 * N_NODES)
    k16 = kmat[0, :16]
    src0 = edge_index[0].astype(jnp.int32)
    dst0 = edge_index[1].astype(jnp.int32)
    loop = jnp.arange(N_NODES, dtype=jnp.int32)
    pad = jnp.zeros((EP - E_TOT,), jnp.int32)
    src_rows = jnp.concatenate([src0, loop, pad]).reshape(EP_ROWS, CHUNK)
    dst_rows = jnp.concatenate([dst0, loop, pad]).reshape(EP_ROWS, CHUNK)
    acc = _edge_pass(_perm_bf16(val0), _perm_bf16(val1), alpha2,
                     src_rows, dst_rows, k16)
    return _norm(acc)


# speedup vs baseline: 1.0187x; 1.0002x over previous
import jax
import jax.numpy as jnp
from jax import lax
from jax.experimental import pallas as pl
from jax.experimental.pallas import tpu as pltpu
from jax.experimental.pallas import tpu_sc as plsc

N_NODES = 10000
N_EDGES = 320000
D = 128
H = 2
DH = 64
E_TOT = N_EDGES + N_NODES

NC = 2
NS = 16
NW = NC * NS
CHUNK = 128
ROWS_PER_TILE = 82
EP_ROWS = NW * ROWS_PER_TILE
EP = EP_ROWS * CHUNK
ACC_W = 80
WB = 624
WB_TAIL = N_NODES - NS * WB


def _prep_body(f_ref, wl_ref, wa_ref, a_ref, val0_ref, val1_ref,
               alpha_ref, k_ref):
    f = f_ref[...]
    val = jnp.dot(f, wl_ref[...].T, preferred_element_type=jnp.float32)
    val0_ref[...] = val[:, 0:DH]
    val1_ref[...] = val[:, DH:2 * DH]
    wa = wa_ref[...]
    av = a_ref[...]
    c0 = jnp.dot(av[0:1, 0:DH], wa[0:DH, :], preferred_element_type=jnp.float32)
    c1 = jnp.dot(av[1:2, 0:DH], wa[DH:2 * DH, :], preferred_element_type=jnp.float32)
    d0 = jnp.dot(av[0:1, DH:2 * DH], wa[0:DH, :], preferred_element_type=jnp.float32)
    d1 = jnp.dot(av[1:2, DH:2 * DH], wa[DH:2 * DH, :], preferred_element_type=jnp.float32)
    cm = jnp.concatenate([c0, c1, d0, d1], axis=0)
    alpha = jnp.dot(f, cm.T, preferred_element_type=jnp.float32)
    alpha_ref[0] = jnp.concatenate([alpha[:, 0:1], alpha[:, 2:3]], axis=1)
    alpha_ref[1] = jnp.concatenate([alpha[:, 1:2], alpha[:, 3:4]], axis=1)
    amax = jnp.max(alpha, axis=0, keepdims=True)
    k0 = jnp.maximum(amax[0, 0] + amax[0, 2], 0.0)
    k1 = jnp.maximum(amax[0, 1] + amax[0, 3], 0.0)
    col = lax.broadcasted_iota(jnp.int32, (8, 128), 1)
    row = lax.broadcasted_iota(jnp.int32, (8, 128), 0)
    k_ref[...] = jnp.where((row == 0) & (col == 0), k0,
                           jnp.where((row == 0) & (col == 1), k1, 0.0))


def _prep(feature, w_lin, w_attn, a):
    return pl.pallas_call(
        _prep_body,
        out_shape=[
            jax.ShapeDtypeStruct((N_NODES, DH), jnp.float32),
            jax.ShapeDtypeStruct((N_NODES, DH), jnp.float32),
            jax.ShapeDtypeStruct((H, N_NODES, 2), jnp.float32),
            jax.ShapeDtypeStruct((8, 128), jnp.float32),
        ],
    )(feature, w_lin, w_attn, a)


def _edge_body(val0_hbm, val1_hbm, alpha_hbm, src_hbm, dst_hbm, k_hbm,
               out_hbm, alpha_v, kv, srcall, dstall, rows_a, rows_b,
               scat_a, scat_b, pb_a, pb_b,
               acc, gsem_a, gsem_b, ssem_a, ssem_b):
    cid = lax.axis_index("c")
    sid = lax.axis_index("s")
    wid = cid * NS + sid

    pltpu.sync_copy(k_hbm, kv)
    pltpu.sync_copy(src_hbm.at[pl.ds(wid * ROWS_PER_TILE, ROWS_PER_TILE)],
                    srcall)
    pltpu.sync_copy(dst_hbm.at[pl.ds(wid * ROWS_PER_TILE, ROWS_PER_TILE)],
                    dstall)
    kvec = kv[pl.ds(0, 16)]
    lane = lax.iota(jnp.int32, 16)

    for h, val_hbm in ((0, val0_hbm), (1, val1_hbm)):
        kh = kvec[h]
        pltpu.sync_copy(alpha_hbm.at[h], alpha_v)

        def _zrow(r, _):
            for c in range(ACC_W // 16):
                scat_a[r, pl.ds(c * 16, 16)] = jnp.zeros((16,), jnp.float32)
            return 0
        lax.fori_loop(0, CHUNK, _zrow, 0)

        for j in range(4):
            pltpu.sync_copy(scat_a, acc.at[pl.ds(sid * WB + j * CHUNK, CHUNK)])
        pltpu.sync_copy(scat_a.at[pl.ds(0, WB - 4 * CHUNK)],
                        acc.at[pl.ds(sid * WB + 4 * CHUNK, WB - 4 * CHUNK)])

        @pl.when(sid == 0)
        def _zero_tail():
            pltpu.sync_copy(scat_a.at[pl.ds(0, WB_TAIL)],
                            acc.at[pl.ds(NS * WB, WB_TAIL)])

        plsc.subcore_barrier()

        def _compute_p(r, pb):
            ebase = (wid * ROWS_PER_TILE + r) * CHUNK

            @plsc.parallel_loop(0, CHUNK // 16, unroll=2)
            def _pgrp(g):
                sv = srcall[r, pl.ds(g * 16, 16)]
                dv = dstall[r, pl.ds(g * 16, 16)]
                a_s = plsc.load_gather(alpha_v, [sv * 2])
                a_d = plsc.load_gather(alpha_v, [dv * 2 + 1])
                s = a_s + a_d
                s = jnp.where(s >= 0, s, 0.2 * s) - kh
                eid = ebase + g * 16 + lane
                valid = (sv != dv) | ((eid >= N_EDGES) & (eid < E_TOT))
                pb[pl.ds(g * 16, 16)] = jnp.where(valid, jnp.exp(s), 0.0)

        def _scale(rows_v, scat_v, pb):
            @plsc.parallel_loop(0, CHUNK // 16, unroll=4)
            def _grp(g2):
                pv = pb[pl.ds(g2 * 16, 16)]
                for j in range(16):
                    e = g2 * 16 + j
                    p = pv[j]
                    ab = rows_v[e, pl.ds(0, 32)]
                    cd = rows_v[e, pl.ds(32, 32)]
                    va, vb = plsc.unpack(
                        ab, format=plsc.PackFormat.INTERLEAVED,
                        preferred_element_type=jnp.float32)
                    vc, vd = plsc.unpack(
                        cd, format=plsc.PackFormat.INTERLEAVED,
                        preferred_element_type=jnp.float32)
                    scat_v[e, pl.ds(0, 16)] = va * p
                    scat_v[e, pl.ds(16, 16)] = vb * p
                    scat_v[e, pl.ds(32, 16)] = vc * p
                    scat_v[e, pl.ds(48, 16)] = vd * p
                    scat_v[e, pl.ds(DH, 16)] = jnp.where(lane == 0, p, 0.0)

        NB = ROWS_PER_TILE // 2

        pltpu.async_copy(val_hbm.at[srcall.at[0]], rows_a, gsem_a)
        pltpu.async_copy(val_hbm.at[srcall.at[1]], rows_b, gsem_b)

        def _iter(i, _):
            r0 = 2 * i
            r1 = 2 * i + 1
            _compute_p(r0, pb_a)
            _compute_p(r1, pb_b)

            pltpu.make_async_copy(
                val_hbm.at[srcall.at[r0]], rows_a, gsem_a).wait()

            @pl.when(i > 0)
            def _drain_a():
                pltpu.make_async_copy(
                    scat_a, acc.at[dstall.at[r0]], ssem_a).wait()

            _scale(rows_a, scat_a, pb_a)
            pltpu.async_copy(scat_a, acc.at[dstall.at[r0]], ssem_a, add=True)

            @pl.when(i < NB - 1)
            def _next_a():
                pltpu.async_copy(
                    val_hbm.at[srcall.at[r0 + 2]], rows_a, gsem_a)

            pltpu.make_async_copy(
                val_hbm.at[srcall.at[r1]], rows_b, gsem_b).wait()

            @pl.when(i > 0)
            def _drain_b():
                pltpu.make_async_copy(
                    scat_b, acc.at[dstall.at[r1]], ssem_b).wait()

            _scale(rows_b, scat_b, pb_b)
            pltpu.async_copy(scat_b, acc.at[dstall.at[r1]], ssem_b, add=True)

            @pl.when(i < NB - 1)
            def _next_b():
                pltpu.async_copy(
                    val_hbm.at[srcall.at[r1 + 2]], rows_b, gsem_b)

            return 0

        lax.fori_loop(0, NB, _iter, 0)
        pltpu.make_async_copy(scat_a, acc.at[dstall.at[0]], ssem_a).wait()
        pltpu.make_async_copy(scat_b, acc.at[dstall.at[1]], ssem_b).wait()
        plsc.subcore_barrier()

        pltpu.sync_copy(acc.at[pl.ds(sid * WB, WB)],
                        out_hbm.at[cid].at[h].at[pl.ds(sid * WB, WB)])

        @pl.when(sid == 0)
        def _tail():
            pltpu.sync_copy(acc.at[pl.ds(NS * WB, WB_TAIL)],
                            out_hbm.at[cid].at[h].at[pl.ds(NS * WB, WB_TAIL)])

        plsc.subcore_barrier()


def _edge_pass(val0, val1, alpha2, src_rows, dst_rows, k16):
    mesh = plsc.VectorSubcoreMesh(core_axis_name="c", subcore_axis_name="s")
    fn = pl.kernel(
        _edge_body,
        out_type=jax.ShapeDtypeStruct((NC, H, N_NODES, ACC_W), jnp.float32),
        mesh=mesh,
        compiler_params=pltpu.CompilerParams(
            use_tc_tiling_on_sc=False, needs_layout_passes=False),
        scratch_types=[
            pltpu.VMEM((N_NODES * 2,), jnp.float32),
            pltpu.VMEM((16,), jnp.float32),
            pltpu.VMEM((ROWS_PER_TILE, CHUNK), jnp.int32),
            pltpu.VMEM((ROWS_PER_TILE, CHUNK), jnp.int32),
            pltpu.VMEM((CHUNK, DH), jnp.bfloat16),
            pltpu.VMEM((CHUNK, DH), jnp.bfloat16),
            pltpu.VMEM((CHUNK, ACC_W), jnp.float32),
            pltpu.VMEM((CHUNK, ACC_W), jnp.float32),
            pltpu.VMEM((CHUNK,), jnp.float32),
            pltpu.VMEM((CHUNK,), jnp.float32),
            pltpu.VMEM_SHARED((N_NODES, ACC_W), jnp.float32),
            pltpu.SemaphoreType.DMA,
            pltpu.SemaphoreType.DMA,
            pltpu.SemaphoreType.DMA,
            pltpu.SemaphoreType.DMA,
        ],
    )
    return fn(val0, val1, alpha2, src_rows, dst_rows, k16)


def _norm_body(acc_ref, out_ref):
    s0 = acc_ref[0, 0] + acc_ref[1, 0]
    s1 = acc_ref[0, 1] + acc_ref[1, 1]
    out_ref[...] = jnp.concatenate(
        [s0[:, 0:DH] / s0[:, DH:DH + 1], s1[:, 0:DH] / s1[:, DH:DH + 1]],
        axis=1)


def _norm(acc):
    return pl.pallas_call(
        _norm_body,
        out_shape=jax.ShapeDtypeStruct((N_NODES, D), jnp.float32),
    )(acc)


def _perm_bf16(v):
    n = v.shape[0]
    return (v.reshape(n, 2, 2, 16).transpose(0, 1, 3, 2)
            .reshape(n, 2 * DH // 2).astype(jnp.bfloat16))


@jax.jit
def kernel(feature, edge_index, W_lin, W_attn, a):
    val0, val1, alpha, kmat = _prep(feature, W_lin, W_attn, a)
    alpha2 = alpha.reshape(H, 2 * N_NODES)
    k16 = kmat[0, :16]
    src0 = edge_index[0].astype(jnp.int32)
    dst0 = edge_index[1].astype(jnp.int32)
    loop = jnp.arange(N_NODES, dtype=jnp.int32)
    pad = jnp.zeros((EP - E_TOT,), jnp.int32)
    src_rows = jnp.concatenate([src0, loop, pad]).reshape(EP_ROWS, CHUNK)
    dst_rows = jnp.concatenate([dst0, loop, pad]).reshape(EP_ROWS, CHUNK)
    acc = _edge_pass(_perm_bf16(val0), _perm_bf16(val1), alpha2,
                     src_rows, dst_rows, k16)
    return _norm(acc)
